# baseline (speedup 1.0000x reference)
import functools

import jax
import jax.numpy as jnp
from jax import lax
from jax.experimental import pallas as pl
from jax.experimental.pallas import tpu as pltpu
from jax.experimental.pallas import tpu_sc as plsc

_NUM_WORKERS = 32
_NS = 4
_NBUF = 2


@functools.partial(jax.jit, static_argnums=(2, 3, 4))
def _embed_lookup(sentence, table, b, s, d):
    sents_per_w = b // _NUM_WORKERS
    n_chunks = sents_per_w // _NS
    n_outer = n_chunks // _NBUF
    mesh = plsc.VectorSubcoreMesh(core_axis_name="c", subcore_axis_name="s")

    @functools.partial(
        pl.kernel,
        mesh=mesh,
        out_type=jax.ShapeDtypeStruct((b, s, d), jnp.float32),
        scratch_types=(
            [pltpu.VMEM((_NS, s), jnp.int32) for _ in range(_NBUF)]
            + [pltpu.VMEM((_NS, s, d), jnp.float32) for _ in range(_NBUF)]
            + [pltpu.SemaphoreType.DMA for _ in range(3 * _NBUF)]
        ),
        compiler_params=pltpu.CompilerParams(use_tc_tiling_on_sc=False),
    )
    def k(sent_hbm, table_hbm, out_hbm, *scratch):
        idx_v = scratch[:_NBUF]
        rows_v = scratch[_NBUF:2 * _NBUF]
        sem_i = scratch[2 * _NBUF:3 * _NBUF]
        sem_g = scratch[3 * _NBUF:4 * _NBUF]
        sem_s = scratch[4 * _NBUF:5 * _NBUF]

        wid = lax.axis_index("s") * 2 + lax.axis_index("c")
        base = wid * sents_per_w

        def idx_load(chunk, bf):
            pltpu.async_copy(
                sent_hbm.at[pl.ds(base + chunk * _NS, _NS), :],
                idx_v[bf], sem_i[bf])

        def gather(bf):
            for j in range(_NS):
                pltpu.async_copy(
                    table_hbm.at[idx_v[bf].at[j]], rows_v[bf].at[j],
                    sem_g[bf])

        def gather_wait(bf):
            for j in range(_NS):
                pltpu.make_async_copy(
                    table_hbm.at[idx_v[bf].at[j]], rows_v[bf].at[j],
                    sem_g[bf]).wait()

        def store(chunk, bf):
            pltpu.async_copy(
                rows_v[bf],
                out_hbm.at[pl.ds(base + chunk * _NS, _NS), :, :],
                sem_s[bf])

        def store_wait(bf):
            pltpu.make_async_copy(
                rows_v[bf],
                out_hbm.at[pl.ds(base, _NS), :, :],
                sem_s[bf]).wait()

        for bf in range(_NBUF):
            idx_load(bf, bf)

        def outer(g, _):
            for bf in range(_NBUF):
                @pl.when(g > 0)
                def _wait_store():
                    store_wait(bf)

                pltpu.make_async_copy(
                    sent_hbm.at[pl.ds(base, _NS), :],
                    idx_v[bf], sem_i[bf]).wait()
                gather(bf)

            for bf in range(_NBUF):
                gather_wait(bf)
                store(g * _NBUF + bf, bf)

                @pl.when(g < n_outer - 1)
                def _next_idx():
                    idx_load((g + 1) * _NBUF + bf, bf)

            return 0

        lax.fori_loop(0, n_outer, outer, 0)

        for bf in range(_NBUF):
            store_wait(bf)

    return k(sentence, table)


def kernel(sentence, table):
    b, s = sentence.shape
    v, d = table.shape
    return _embed_lookup(sentence.astype(jnp.int32), table, b, s, d)


# speedup vs baseline: 1.3061x; 1.3061x over previous
import functools

import jax
import jax.numpy as jnp
from jax import lax
from jax.experimental import pallas as pl
from jax.experimental.pallas import tpu as pltpu
from jax.experimental.pallas import tpu_sc as plsc

_NUM_WORKERS = 32
_GRP = 8


@functools.partial(jax.jit, static_argnums=(2, 3, 4))
def _embed_lookup(sentence, table, b, s, d):
    sents_per_w = b // _NUM_WORKERS
    n_grps = sents_per_w // _GRP
    mesh = plsc.VectorSubcoreMesh(core_axis_name="c", subcore_axis_name="s")

    @functools.partial(
        pl.kernel,
        mesh=mesh,
        out_type=jax.ShapeDtypeStruct((b, s, d), jnp.float32),
        scratch_types=(
            [pltpu.VMEM((_GRP, s), jnp.int32)]
            + [pltpu.VMEM((s, d), jnp.float32) for _ in range(2)]
            + [pltpu.SemaphoreType.DMA for _ in range(4)]
        ),
    )
    def k(sent_hbm, table_hbm, out_hbm, idx_v, rows0, rows1,
          gs0, gs1, ss0, ss1):
        rows_v = (rows0, rows1)
        sem_g = (gs0, gs1)
        sem_s = (ss0, ss1)

        wid = lax.axis_index("s") * 2 + lax.axis_index("c")
        base = wid * sents_per_w

        def grp_body(g, _):
            s0 = base + g * _GRP
            pltpu.sync_copy(sent_hbm.at[pl.ds(s0, _GRP), :], idx_v)
            for j in range(_GRP):
                bf = j % 2
                sent = s0 + j

                def issue_rows():
                    def issue16(off, lanes):
                        v = idx_v[j, pl.ds(off, 16)]
                        for l in lanes:
                            pltpu.async_copy(
                                table_hbm.at[pl.ds(v[l], 1), :],
                                rows_v[bf].at[pl.ds(off + l, 1), :],
                                sem_g[bf])

                    def blk_body(t16, _):
                        issue16(t16 * 16, range(16))
                        return 0
                    n_full = s // 16
                    lax.fori_loop(0, n_full, blk_body, 0)
                    if s % 16:
                        issue16(s - 16, range(16 - s % 16, 16))

                if j >= 2:
                    pltpu.make_async_copy(
                        rows_v[bf], out_hbm.at[base], sem_s[bf]).wait()
                    issue_rows()
                else:
                    @pl.when(g > 0)
                    def _drain_prev():
                        pltpu.make_async_copy(
                            rows_v[bf], out_hbm.at[base], sem_s[bf]).wait()
                    issue_rows()

                pltpu.make_async_copy(
                    out_hbm.at[sent], rows_v[bf], sem_g[bf]).wait()
                pltpu.async_copy(rows_v[bf], out_hbm.at[sent], sem_s[bf])
            return 0

        lax.fori_loop(0, n_grps, grp_body, 0)

        for bf in range(2):
            pltpu.make_async_copy(
                rows_v[bf], out_hbm.at[base], sem_s[bf]).wait()

    return k(sentence, table)


def kernel(sentence, table):
    b, s = sentence.shape
    v, d = table.shape
    return _embed_lookup(sentence.astype(jnp.int32), table, b, s, d)
